# baseline (speedup 1.0000x reference)
import functools
import jax
import jax.numpy as jnp
from jax import lax
from jax.experimental import pallas as pl
from jax.experimental.pallas import tpu as pltpu
from jax.experimental.pallas import tpu_sc as plsc

_L = 16


def _make_sc_kernel(B, S, D):
    NC, NS = 2, 16
    NW = NC * NS
    SPW = S // NW
    CHUNK = 16
    NCHUNK = SPW // CHUNK
    NSTEP = NCHUNK * B
    COLS = D // _L
    mesh = plsc.VectorSubcoreMesh(core_axis_name="c", subcore_axis_name="s")

    @functools.partial(
        pl.kernel,
        mesh=mesh,
        out_type=jax.ShapeDtypeStruct((B, S, D), jnp.float32),
        scratch_types=[
            pltpu.VMEM((2, CHUNK, D), jnp.float32),
            pltpu.VMEM((2, CHUNK, D), jnp.float32),
            pltpu.VMEM((CHUNK, D), jnp.float32),
            pltpu.SemaphoreType.DMA,
            pltpu.SemaphoreType.DMA,
            pltpu.SemaphoreType.DMA,
            pltpu.SemaphoreType.DMA,
        ],
    )
    def k(x_hbm, pos_hbm, out_hbm, xin, xout, pbuf, isem0, isem1, osem0, osem1):
        wid = lax.axis_index("s") * NC + lax.axis_index("c")
        base = wid * SPW
        isems = (isem0, isem1)
        osems = (osem0, osem1)

        def rows_of(step):
            return base + (step // B) * CHUNK

        def batch_of(step):
            return step % B

        def start_in(step, slot):
            pltpu.async_copy(
                x_hbm.at[batch_of(step), pl.ds(rows_of(step), CHUNK)],
                xin.at[slot],
                isems[slot],
            )

        start_in(0, 0)
        pltpu.sync_copy(pos_hbm.at[pl.ds(base, CHUNK)], pbuf)

        def half_step(step, slot):
            @pl.when(jnp.logical_and(step > 0, batch_of(step) == 0))
            def _():
                pltpu.sync_copy(pos_hbm.at[pl.ds(rows_of(step), CHUNK)], pbuf)

            @pl.when(step + 1 < NSTEP)
            def _():
                start_in(step + 1, 1 - slot)

            pltpu.make_async_copy(
                x_hbm.at[0, pl.ds(0, CHUNK)], xin.at[slot], isems[slot]
            ).wait()

            @pl.when(step >= 2)
            def _():
                pltpu.make_async_copy(
                    xout.at[slot], out_hbm.at[0, pl.ds(0, CHUNK)], osems[slot]
                ).wait()

            for r in range(CHUNK):
                def col_body(j, acc):
                    for u in range(4):
                        c = j * (4 * _L) + u * _L
                        xout[slot, r, pl.ds(c, _L)] = (
                            xin[slot, r, pl.ds(c, _L)] + pbuf[r, pl.ds(c, _L)]
                        )
                    return acc

                lax.fori_loop(0, COLS // 4, col_body, 0)

            pltpu.async_copy(
                xout.at[slot],
                out_hbm.at[batch_of(step), pl.ds(rows_of(step), CHUNK)],
                osems[slot],
            )

        def step_body(g, carry):
            half_step(2 * g, 0)
            half_step(2 * g + 1, 1)
            return carry

        lax.fori_loop(0, NSTEP // 2, step_body, 0)

        for slot in range(2):
            pltpu.make_async_copy(
                xout.at[slot], out_hbm.at[0, pl.ds(0, CHUNK)], osems[slot]
            ).wait()

    return k


def kernel(x, pos_emb):
    B, S, D = x.shape
    return _make_sc_kernel(B, S, D)(x, pos_emb)


# speedup vs baseline: 3.7590x; 3.7590x over previous
import jax
import jax.numpy as jnp
from jax.experimental import pallas as pl
from jax.experimental.pallas import tpu as pltpu


def _add_kernel(x_ref, p_ref, o_ref):
    o_ref[...] = x_ref[...] + p_ref[...]


def kernel(x, pos_emb):
    B, S, D = x.shape
    SBLK = 2048
    return pl.pallas_call(
        _add_kernel,
        grid=(S // SBLK, B),
        in_specs=[
            pl.BlockSpec((1, SBLK, D), lambda s, b: (b, s, 0)),
            pl.BlockSpec((SBLK, D), lambda s, b: (s, 0)),
        ],
        out_specs=pl.BlockSpec((1, SBLK, D), lambda s, b: (b, s, 0)),
        out_shape=jax.ShapeDtypeStruct((B, S, D), x.dtype),
        compiler_params=pltpu.CompilerParams(
            dimension_semantics=("parallel", "parallel"),
        ),
    )(x, pos_emb)
